# baseline (speedup 1.0000x reference)
import functools

import jax
import jax.numpy as jnp
from jax import lax
from jax.experimental import pallas as pl
from jax.experimental.pallas import tpu as pltpu


def _hidden_bn_relu_kernel(x_ref, w1_ref, g_ref, beta_ref, hn_ref, *, eps):
    h = jnp.dot(x_ref[...], w1_ref[...], preferred_element_type=jnp.float32)
    mean = jnp.mean(h, axis=0, keepdims=True)
    cent = h - mean
    var = jnp.mean(cent * cent, axis=0, keepdims=True)
    scale = g_ref[...] * lax.rsqrt(var + eps)
    hn = jnp.maximum(cent * scale + beta_ref[...], 0.0)
    hn_ref[...] = hn.astype(hn_ref.dtype)


def _out_proj_kernel(hn_ref, w2_ref, b2_ref, o_ref):
    o_ref[...] = (jnp.dot(hn_ref[...], w2_ref[...],
                          preferred_element_type=jnp.float32)
                  + b2_ref[...])


def kernel(w1, b1, gamma, beta, w2, b2, x):
    del b1
    B, in_dim = x.shape
    hidden = w1.shape[1]
    out_dim = w2.shape[1]
    eps = 1e-5

    xm = x.astype(jnp.bfloat16)
    w1m = w1.astype(jnp.bfloat16)
    w2m = w2.astype(jnp.bfloat16)
    g2 = gamma.reshape(1, hidden).astype(jnp.float32)
    beta2 = beta.reshape(1, hidden).astype(jnp.float32)
    b2_2 = b2.reshape(1, out_dim).astype(jnp.float32)

    nh = 2 if hidden % 256 == 0 else 1
    th = hidden // nh
    hn = pl.pallas_call(
        functools.partial(_hidden_bn_relu_kernel, eps=eps),
        grid=(nh,),
        in_specs=[
            pl.BlockSpec((B, in_dim), lambda i: (0, 0)),
            pl.BlockSpec((in_dim, th), lambda i: (0, i)),
            pl.BlockSpec((1, th), lambda i: (0, i)),
            pl.BlockSpec((1, th), lambda i: (0, i)),
        ],
        out_specs=pl.BlockSpec((B, th), lambda i: (0, i)),
        out_shape=jax.ShapeDtypeStruct((B, hidden), jnp.bfloat16),
        compiler_params=pltpu.CompilerParams(
            dimension_semantics=("parallel",)),
        cost_estimate=pl.CostEstimate(
            flops=2 * B * in_dim * hidden,
            transcendentals=hidden,
            bytes_accessed=(2 * B * in_dim + in_dim * hidden + B * hidden) * 2
            + 2 * hidden * 4,
        ),
    )(xm, w1m, g2, beta2)

    nn = 2 if out_dim % 256 == 0 else 1
    tn = out_dim // nn
    return pl.pallas_call(
        _out_proj_kernel,
        grid=(nn,),
        in_specs=[
            pl.BlockSpec((B, hidden), lambda j: (0, 0)),
            pl.BlockSpec((hidden, tn), lambda j: (0, j)),
            pl.BlockSpec((1, tn), lambda j: (0, j)),
        ],
        out_specs=pl.BlockSpec((B, tn), lambda j: (0, j)),
        out_shape=jax.ShapeDtypeStruct((B, out_dim), jnp.float32),
        compiler_params=pltpu.CompilerParams(
            dimension_semantics=("parallel",)),
        cost_estimate=pl.CostEstimate(
            flops=2 * B * hidden * out_dim,
            transcendentals=0,
            bytes_accessed=(2 * B * hidden + hidden * out_dim) * 2
            + (out_dim + B * out_dim) * 4,
        ),
    )(hn, w2m, b2_2)


# speedup vs baseline: 1.2992x; 1.2992x over previous
import functools

import jax
import jax.numpy as jnp
from jax import lax
from jax.experimental import pallas as pl
from jax.experimental.pallas import tpu as pltpu


def _hidden_bn_relu_kernel(x_ref, w1_ref, g_ref, beta_ref, hn_ref, *, eps,
                           inv_b):
    h = jnp.dot(x_ref[...], w1_ref[...], preferred_element_type=jnp.float32)
    mean = jnp.sum(h, axis=0, keepdims=True) * inv_b
    var = jnp.sum(h * h, axis=0, keepdims=True) * inv_b - mean * mean
    a = g_ref[...] * lax.rsqrt(jnp.maximum(var, 0.0) + eps)
    c = beta_ref[...] - mean * a
    hn_ref[...] = jnp.maximum(h * a + c, 0.0).astype(hn_ref.dtype)


def _out_proj_kernel(hn_ref, w2_ref, b2_ref, o_ref):
    k = pl.program_id(1)

    @pl.when(k == 0)
    def _init():
        o_ref[...] = jnp.broadcast_to(b2_ref[...], o_ref.shape)

    o_ref[...] += jnp.dot(hn_ref[...], w2_ref[...].astype(hn_ref.dtype),
                          preferred_element_type=jnp.float32)


def kernel(w1, b1, gamma, beta, w2, b2, x):
    del b1
    B, in_dim = x.shape
    hidden = w1.shape[1]
    out_dim = w2.shape[1]
    eps = 1e-5

    g2 = gamma.reshape(1, hidden)
    beta2 = beta.reshape(1, hidden)
    b2_2 = b2.reshape(1, out_dim)

    th = 256 if hidden % 512 == 0 else hidden
    nh = hidden // th
    ncore = 2 if nh % 2 == 0 else 1
    nj = nh // ncore
    hn = pl.pallas_call(
        functools.partial(_hidden_bn_relu_kernel, eps=eps, inv_b=1.0 / B),
        grid=(ncore, nj),
        in_specs=[
            pl.BlockSpec((B, in_dim), lambda i, j: (0, 0)),
            pl.BlockSpec((in_dim, th), lambda i, j: (0, i * nj + j)),
            pl.BlockSpec((1, th), lambda i, j: (0, i * nj + j)),
            pl.BlockSpec((1, th), lambda i, j: (0, i * nj + j)),
        ],
        out_specs=pl.BlockSpec((B, th), lambda i, j: (0, i * nj + j)),
        out_shape=jax.ShapeDtypeStruct((B, hidden), jnp.bfloat16),
        compiler_params=pltpu.CompilerParams(
            dimension_semantics=("parallel", "arbitrary")),
        cost_estimate=pl.CostEstimate(
            flops=2 * B * in_dim * hidden,
            transcendentals=hidden,
            bytes_accessed=(2 * B * in_dim + in_dim * hidden) * 4
            + B * hidden * 2 + 2 * hidden * 4,
        ),
    )(x, w1, g2, beta2)

    tn = out_dim // 2 if out_dim % 256 == 0 else out_dim
    nn = out_dim // tn
    tk = 512 if hidden % 512 == 0 else hidden
    nk = hidden // tk
    return pl.pallas_call(
        _out_proj_kernel,
        grid=(nn, nk),
        in_specs=[
            pl.BlockSpec((B, tk), lambda j, k: (0, k)),
            pl.BlockSpec((tk, tn), lambda j, k: (k, j)),
            pl.BlockSpec((1, tn), lambda j, k: (0, j)),
        ],
        out_specs=pl.BlockSpec((B, tn), lambda j, k: (0, j)),
        out_shape=jax.ShapeDtypeStruct((B, out_dim), jnp.float32),
        compiler_params=pltpu.CompilerParams(
            dimension_semantics=("parallel", "arbitrary")),
        cost_estimate=pl.CostEstimate(
            flops=2 * B * hidden * out_dim,
            transcendentals=0,
            bytes_accessed=B * hidden * 2 + hidden * out_dim * 4
            + (out_dim + B * out_dim) * 4,
        ),
    )(hn, w2, b2_2)
